# baseline (speedup 1.0000x reference)
import functools

import jax
import jax.numpy as jnp
from jax.experimental import pallas as pl
from jax.experimental.pallas import tpu as pltpu

B = 32
N = 4096
S = 1024
BBLK = 16


def _fps_body(x_ref, y_ref, z_ref, out_ref):
    x = x_ref[...]
    y = y_ref[...]
    z = z_ref[...]
    iota = jax.lax.broadcasted_iota(jnp.int32, (BBLK, N), 1)
    lane_iota = jax.lax.broadcasted_iota(jnp.int32, (BBLK, 128), 1)

    cx0 = x[:, 0:1]
    cy0 = y[:, 0:1]
    cz0 = z[:, 0:1]
    dists0 = jnp.full((BBLK, N), jnp.inf, dtype=jnp.float32)

    def body(l, carry):
        dists, cx, cy, cz, buf = carry
        dx = x - cx
        dy = y - cy
        dz = z - cz
        d = (dx * dx + dz * dz) + dy * dy
        dists = jnp.minimum(dists, d)
        m = jnp.max(dists, axis=1, keepdims=True)
        idx = jnp.min(jnp.where(dists == m, iota, N), axis=1, keepdims=True)
        sel = iota == idx
        cx = jnp.sum(jnp.where(sel, x, 0.0), axis=1, keepdims=True)
        cy = jnp.sum(jnp.where(sel, y, 0.0), axis=1, keepdims=True)
        cz = jnp.sum(jnp.where(sel, z, 0.0), axis=1, keepdims=True)
        buf = jnp.where(lane_iota == l, idx, buf)
        return dists, cx, cy, cz, buf

    state = (dists0, cx0, cy0, cz0)
    for c in range(S // 128):
        buf0 = jnp.zeros((BBLK, 128), jnp.int32)
        start = 1 if c == 0 else 0
        *state, buf = jax.lax.fori_loop(start, 128, body, (*state, buf0))
        out_ref[:, c * 128 : (c + 1) * 128] = buf
        state = tuple(state)


@jax.jit
def kernel(input):
    pts = input
    x = pts[:, :, 0]
    y = pts[:, :, 1]
    z = pts[:, :, 2]
    coord_spec = pl.BlockSpec((BBLK, N), lambda i: (i, 0))
    out = pl.pallas_call(
        _fps_body,
        grid=(B // BBLK,),
        in_specs=[coord_spec, coord_spec, coord_spec],
        out_specs=pl.BlockSpec((BBLK, S), lambda i: (i, 0)),
        out_shape=jax.ShapeDtypeStruct((B, S), jnp.int32),
        compiler_params=pltpu.CompilerParams(
            dimension_semantics=("parallel",),
        ),
    )(x, y, z)
    return out


# speedup vs baseline: 1.1504x; 1.1504x over previous
import numpy as np

import jax
import jax.numpy as jnp
from jax.experimental import pallas as pl
from jax.experimental import shard_map
from jax.sharding import Mesh, PartitionSpec as P

B = 32
N = 4096
S = 1024


def _fps_body(x_ref, y_ref, z_ref, out_ref):
    bblk = x_ref.shape[0]
    x = x_ref[...]
    y = y_ref[...]
    z = z_ref[...]
    iota = jax.lax.broadcasted_iota(jnp.int32, (bblk, N), 1)
    lane_iota = jax.lax.broadcasted_iota(jnp.int32, (bblk, 128), 1)

    cx0 = x[:, 0:1]
    cy0 = y[:, 0:1]
    cz0 = z[:, 0:1]
    dists0 = jnp.full((bblk, N), jnp.inf, dtype=jnp.float32)

    def body(l, carry):
        dists, cx, cy, cz, buf = carry
        dx = x - cx
        dy = y - cy
        dz = z - cz
        d = (dx * dx + dz * dz) + dy * dy
        dists = jnp.minimum(dists, d)
        m = jnp.max(dists, axis=1, keepdims=True)
        idx = jnp.min(jnp.where(dists == m, iota, N), axis=1, keepdims=True)
        sel = iota == idx
        cx = jnp.sum(jnp.where(sel, x, 0.0), axis=1, keepdims=True)
        cy = jnp.sum(jnp.where(sel, y, 0.0), axis=1, keepdims=True)
        cz = jnp.sum(jnp.where(sel, z, 0.0), axis=1, keepdims=True)
        buf = jnp.where(lane_iota == l, idx, buf)
        return dists, cx, cy, cz, buf

    state = (dists0, cx0, cy0, cz0)
    for c in range(S // 128):
        buf0 = jnp.zeros((bblk, 128), jnp.int32)
        start = 1 if c == 0 else 0
        *state, buf = jax.lax.fori_loop(start, 128, body, (*state, buf0))
        out_ref[:, c * 128 : (c + 1) * 128] = buf
        state = tuple(state)


def _fps_call(x, y, z):
    bblk = x.shape[0]
    return pl.pallas_call(
        _fps_body,
        out_shape=jax.ShapeDtypeStruct((bblk, S), jnp.int32),
    )(x, y, z)


@jax.jit
def kernel(input):
    pts = input
    x = pts[:, :, 0]
    y = pts[:, :, 1]
    z = pts[:, :, 2]
    devs = jax.devices()
    ndev = 2 if len(devs) >= 2 else 1
    if ndev == 1:
        return _fps_call(x, y, z)
    mesh = Mesh(np.array(devs[:ndev]), ("d",))
    f = shard_map.shard_map(
        _fps_call,
        mesh=mesh,
        in_specs=(P("d"), P("d"), P("d")),
        out_specs=P("d"),
        check_rep=False,
    )
    return f(x, y, z)


# speedup vs baseline: 1.6126x; 1.4018x over previous
import numpy as np

import jax
import jax.numpy as jnp
from jax.experimental import pallas as pl
from jax.experimental import shard_map
from jax.sharding import Mesh, PartitionSpec as P

B = 32
N = 4096
S = 1024


def _fps_body(x_ref, y_ref, z_ref, out_ref):
    bblk = x_ref.shape[0]
    x = x_ref[...]
    y = y_ref[...]
    z = z_ref[...]
    iota = jax.lax.broadcasted_iota(jnp.int32, (bblk, N), 1)
    lane_iota = jax.lax.broadcasted_iota(jnp.int32, (bblk, 128), 1)

    cx0 = x[:, 0:1]
    cy0 = y[:, 0:1]
    cz0 = z[:, 0:1]
    dists0 = jnp.full((bblk, N), jnp.inf, dtype=jnp.float32)

    def body(l, carry):
        dists, cx, cy, cz, buf = carry
        dx = x - cx
        dy = y - cy
        dz = z - cz
        d = (dx * dx + dz * dz) + dy * dy
        dists = jnp.minimum(dists, d)
        idx = jnp.argmax(dists, axis=1, keepdims=True).astype(jnp.int32)
        selm = jnp.where(iota == idx, 1.0, 0.0)
        cx = jnp.sum(x * selm, axis=1, keepdims=True)
        cy = jnp.sum(y * selm, axis=1, keepdims=True)
        cz = jnp.sum(z * selm, axis=1, keepdims=True)
        buf = jnp.where(lane_iota == l, idx, buf)
        return dists, cx, cy, cz, buf

    state = (dists0, cx0, cy0, cz0)
    for c in range(S // 128):
        buf0 = jnp.zeros((bblk, 128), jnp.int32)
        start = 1 if c == 0 else 0
        *state, buf = jax.lax.fori_loop(start, 128, body, (*state, buf0))
        out_ref[:, c * 128 : (c + 1) * 128] = buf
        state = tuple(state)


def _fps_call(x, y, z):
    bblk = x.shape[0]
    return pl.pallas_call(
        _fps_body,
        out_shape=jax.ShapeDtypeStruct((bblk, S), jnp.int32),
    )(x, y, z)


@jax.jit
def kernel(input):
    pts = input
    x = pts[:, :, 0]
    y = pts[:, :, 1]
    z = pts[:, :, 2]
    return _fps_call(x, y, z)


# speedup vs baseline: 2.0058x; 1.2438x over previous
import numpy as np

import jax
import jax.numpy as jnp
from jax.experimental import pallas as pl
from jax.experimental import shard_map
from jax.sharding import Mesh, PartitionSpec as P

B = 32
N = 4096
S = 1024


def _fps_body(x_ref, y_ref, z_ref, out_ref):
    bblk = x_ref.shape[0]
    x = x_ref[...]
    y = y_ref[...]
    z = z_ref[...]
    iota = jax.lax.broadcasted_iota(jnp.int32, (bblk, N), 1)
    lane_iota = jax.lax.broadcasted_iota(jnp.int32, (bblk, 128), 1)

    cx0 = x[:, 0:1]
    cy0 = y[:, 0:1]
    cz0 = z[:, 0:1]
    dists0 = jnp.full((bblk, N), jnp.inf, dtype=jnp.float32)

    def body(l, carry):
        dists, cx, cy, cz, buf = carry
        dx = x - cx
        dy = y - cy
        dz = z - cz
        d = (dx * dx + dz * dz) + dy * dy
        dists = jnp.minimum(dists, d)
        idx = jnp.argmax(dists, axis=1, keepdims=True).astype(jnp.int32)
        selm = jnp.where(iota == idx, 1.0, 0.0)
        cx = jnp.sum(x * selm, axis=1, keepdims=True)
        cy = jnp.sum(y * selm, axis=1, keepdims=True)
        cz = jnp.sum(z * selm, axis=1, keepdims=True)
        buf = jnp.where(lane_iota == l, idx, buf)
        return dists, cx, cy, cz, buf

    state = (dists0, cx0, cy0, cz0)
    for c in range(S // 128):
        buf0 = jnp.zeros((bblk, 128), jnp.int32)
        start = 1 if c == 0 else 0
        *state, buf = jax.lax.fori_loop(start, 128, body, (*state, buf0), unroll=2)
        out_ref[:, c * 128 : (c + 1) * 128] = buf
        state = tuple(state)


def _fps_call(x, y, z):
    bblk = x.shape[0]
    return pl.pallas_call(
        _fps_body,
        out_shape=jax.ShapeDtypeStruct((bblk, S), jnp.int32),
    )(x, y, z)


@jax.jit
def kernel(input):
    pts = input
    x = pts[:, :, 0]
    y = pts[:, :, 1]
    z = pts[:, :, 2]
    return _fps_call(x, y, z)


# speedup vs baseline: 2.1328x; 1.0634x over previous
import numpy as np

import jax
import jax.numpy as jnp
from jax.experimental import pallas as pl
from jax.experimental import shard_map
from jax.sharding import Mesh, PartitionSpec as P

B = 32
N = 4096
S = 1024


def _fps_body(x_ref, y_ref, z_ref, out_ref):
    bblk = x_ref.shape[0]
    x = x_ref[...]
    y = y_ref[...]
    z = z_ref[...]
    iota = jax.lax.broadcasted_iota(jnp.int32, (bblk, N), 1)
    lane_iota = jax.lax.broadcasted_iota(jnp.int32, (bblk, 128), 1)

    cx0 = x[:, 0:1]
    cy0 = y[:, 0:1]
    cz0 = z[:, 0:1]
    dists0 = jnp.full((bblk, N), jnp.inf, dtype=jnp.float32)

    def body(l, carry):
        dists, cx, cy, cz, buf = carry
        dx = x - cx
        dy = y - cy
        dz = z - cz
        d = (dx * dx + dz * dz) + dy * dy
        dists = jnp.minimum(dists, d)
        idx = jnp.argmax(dists, axis=1, keepdims=True).astype(jnp.int32)
        selm = jnp.where(iota == idx, 1.0, 0.0)
        cx = jnp.sum(x * selm, axis=1, keepdims=True)
        cy = jnp.sum(y * selm, axis=1, keepdims=True)
        cz = jnp.sum(z * selm, axis=1, keepdims=True)
        buf = jnp.where(lane_iota == l, idx, buf)
        return dists, cx, cy, cz, buf

    state = (dists0, cx0, cy0, cz0)
    for c in range(S // 128):
        buf0 = jnp.zeros((bblk, 128), jnp.int32)
        start = 1 if c == 0 else 0
        *state, buf = jax.lax.fori_loop(start, 128, body, (*state, buf0), unroll=4)
        out_ref[:, c * 128 : (c + 1) * 128] = buf
        state = tuple(state)


def _fps_call(x, y, z):
    bblk = x.shape[0]
    return pl.pallas_call(
        _fps_body,
        out_shape=jax.ShapeDtypeStruct((bblk, S), jnp.int32),
    )(x, y, z)


@jax.jit
def kernel(input):
    pts = input
    x = pts[:, :, 0]
    y = pts[:, :, 1]
    z = pts[:, :, 2]
    return _fps_call(x, y, z)
